# baseline (device time: 177270 ns/iter reference)
import jax
import jax.numpy as jnp
from jax import lax
from jax.experimental import pallas as pl
from jax.experimental.pallas import tpu as pltpu

B, SQ, H, D = 4, 32, 8, 128
SKV = 4096
KC = 512
NKC = SKV // KC
N_DEV = 4
ROWS = B * H * SQ
PACKED_ROWS = ROWS + ROWS // D
SCALE = D ** -0.5


def _flash_partial_body(
    q_ref, k_hbm, v_hbm, a_ref, r_ref,
    kraw, vraw, kbuf, vbuf,
    kraw_sems, vraw_sems, ksems, vsems, a_acc, r_acc,
):
    b = pl.program_id(0)
    kc = pl.program_id(1)
    t = b * NKC + kc
    g = lax.rem(t, 2)

    def raw_issue(c):
        slot = lax.rem(c, 2)
        bb = c // NKC
        base = lax.rem(c, NKC) * KC
        pltpu.make_async_copy(
            k_hbm.at[bb, pl.ds(base, KC), :, :], kraw.at[slot], kraw_sems.at[slot]
        ).start()
        pltpu.make_async_copy(
            v_hbm.at[bb, pl.ds(base, KC), :, :], vraw.at[slot], vraw_sems.at[slot]
        ).start()

    def raw_wait(c):
        slot = lax.rem(c, 2)
        pltpu.make_async_copy(
            k_hbm.at[0, pl.ds(0, KC), :, :], kraw.at[slot], kraw_sems.at[slot]
        ).wait()
        pltpu.make_async_copy(
            v_hbm.at[0, pl.ds(0, KC), :, :], vraw.at[slot], vraw_sems.at[slot]
        ).wait()

    def gather_issue(c):
        slot = lax.rem(c, 2)
        for h in range(H):
            pltpu.make_async_copy(
                kraw.at[slot, :, h, :], kbuf.at[slot, h], ksems.at[slot, h]
            ).start()
            pltpu.make_async_copy(
                vraw.at[slot, :, h, :], vbuf.at[slot, h], vsems.at[slot, h]
            ).start()

    @pl.when(t == 0)
    def _():
        raw_issue(0)
        raw_issue(1)
        raw_wait(0)
        gather_issue(0)

    for h in range(H):
        pltpu.make_async_copy(
            kraw.at[g, :, h, :], kbuf.at[g, h], ksems.at[g, h]
        ).wait()
        pltpu.make_async_copy(
            vraw.at[g, :, h, :], vbuf.at[g, h], vsems.at[g, h]
        ).wait()

    @pl.when(t + 1 < B * NKC)
    def _():
        raw_wait(t + 1)
        gather_issue(t + 1)

    @pl.when(t + 2 < B * NKC)
    def _():
        raw_issue(t + 2)

    @pl.when(kc == 0)
    def _():
        a_acc[...] = jnp.zeros_like(a_acc)
        r_acc[...] = jnp.zeros_like(r_acc)

    for h in range(H):
        q = (q_ref[0, :, h, :] * SCALE).astype(jnp.bfloat16)
        k = kbuf[g, h].astype(jnp.bfloat16)
        v = vbuf[g, h].astype(jnp.bfloat16)

        s = lax.dot_general(
            q, k, (((1,), (1,)), ((), ())),
            preferred_element_type=jnp.float32,
        )
        p = jnp.exp(s)
        r_acc[h, :, :] += jnp.sum(p, axis=1, keepdims=True)
        a_acc[h, :, :] += lax.dot_general(
            p.astype(jnp.bfloat16), v, (((1,), (0,)), ((), ())),
            preferred_element_type=jnp.float32,
        )

    @pl.when(kc == NKC - 1)
    def _():
        a_ref[0, :, :, :] = a_acc[...]
        r_ref[0, :, :, :] = r_acc[...]


def _flash_partial(Q, K, V):
    return pl.pallas_call(
        _flash_partial_body,
        grid=(B, NKC),
        in_specs=[
            pl.BlockSpec((1, SQ, H, D), lambda b, k: (b, 0, 0, 0)),
            pl.BlockSpec(memory_space=pltpu.MemorySpace.HBM),
            pl.BlockSpec(memory_space=pltpu.MemorySpace.HBM),
        ],
        out_specs=[
            pl.BlockSpec((1, H, SQ, D), lambda b, k: (b, 0, 0, 0)),
            pl.BlockSpec((1, H, SQ, 1), lambda b, k: (b, 0, 0, 0)),
        ],
        out_shape=[
            jax.ShapeDtypeStruct((B, H, SQ, D), jnp.float32),
            jax.ShapeDtypeStruct((B, H, SQ, 1), jnp.float32),
        ],
        scratch_shapes=[
            pltpu.VMEM((2, KC, H, D), jnp.float32),
            pltpu.VMEM((2, KC, H, D), jnp.float32),
            pltpu.VMEM((2, H, KC, D), jnp.float32),
            pltpu.VMEM((2, H, KC, D), jnp.float32),
            pltpu.SemaphoreType.DMA((2,)),
            pltpu.SemaphoreType.DMA((2,)),
            pltpu.SemaphoreType.DMA((2, H)),
            pltpu.SemaphoreType.DMA((2, H)),
            pltpu.VMEM((H, SQ, D), jnp.float32),
            pltpu.VMEM((H, SQ, 1), jnp.float32),
        ],
        compiler_params=pltpu.CompilerParams(
            dimension_semantics=("arbitrary", "arbitrary"),
        ),
    )(Q, K, V)


def _allreduce_body(x_ref, out_ref, comm_ref, send_buf, send_sems, recv_sems):
    my = lax.axis_index("i")
    p1 = my ^ 1
    p2 = 3 - my

    barrier_sem = pltpu.get_barrier_semaphore()
    for nbr in (p1, p2):
        pl.semaphore_signal(
            barrier_sem, inc=1,
            device_id=(nbr,), device_id_type=pl.DeviceIdType.MESH,
        )
    pl.semaphore_wait(barrier_sem, 2)

    rdma1 = pltpu.make_async_remote_copy(
        src_ref=x_ref,
        dst_ref=comm_ref.at[0],
        send_sem=send_sems.at[0],
        recv_sem=recv_sems.at[0],
        device_id=(p1,),
        device_id_type=pl.DeviceIdType.MESH,
    )
    rdma1.start()
    rdma1.wait()

    part = x_ref[...].astype(jnp.float32) + comm_ref[0].astype(jnp.float32)
    send_buf[...] = part.astype(jnp.bfloat16)

    rdma2 = pltpu.make_async_remote_copy(
        src_ref=send_buf,
        dst_ref=comm_ref.at[1],
        send_sem=send_sems.at[1],
        recv_sem=recv_sems.at[1],
        device_id=(p2,),
        device_id_type=pl.DeviceIdType.MESH,
    )
    rdma2.start()
    rdma2.wait()

    out_ref[...] = part + comm_ref[1].astype(jnp.float32)


def _allreduce(packed):
    return pl.pallas_call(
        _allreduce_body,
        in_specs=[pl.BlockSpec(memory_space=pltpu.VMEM)],
        out_specs=pl.BlockSpec(memory_space=pltpu.VMEM),
        out_shape=jax.ShapeDtypeStruct((PACKED_ROWS, D), jnp.float32),
        scratch_shapes=[
            pltpu.VMEM((2, PACKED_ROWS, D), jnp.bfloat16),
            pltpu.VMEM((PACKED_ROWS, D), jnp.bfloat16),
            pltpu.SemaphoreType.DMA((2,)),
            pltpu.SemaphoreType.DMA((2,)),
        ],
        compiler_params=pltpu.CompilerParams(collective_id=0),
    )(packed)


def kernel(Q, K, V):
    A, r = _flash_partial(Q, K, V)
    packed = jnp.concatenate(
        [A.reshape(ROWS, D), r.reshape(ROWS // D, D)], axis=0
    ).astype(jnp.bfloat16)
    red = _allreduce(packed)
    o = red[:ROWS] / red[ROWS:].reshape(ROWS, 1)
    return o.reshape(B, H, SQ, D).transpose(0, 2, 1, 3)


# device time: 63937 ns/iter; 2.7726x vs baseline; 2.7726x over previous
import jax
import jax.numpy as jnp
from jax import lax
from jax.experimental import pallas as pl
from jax.experimental.pallas import tpu as pltpu

B, SQ, H, D = 4, 32, 8, 128
SKV = 4096
KC = 1024
NKC = SKV // KC
N_DEV = 4
ROWS = B * H * SQ
PACKED_ROWS = ROWS + ROWS // D
SCALE = D ** -0.5


def _flash_partial_body(
    q_ref, k_hbm, v_hbm, a_ref, r_ref,
    kbuf, vbuf, ksems, vsems, a_acc, r_acc,
):
    b = pl.program_id(0)
    kc = pl.program_id(1)
    slot = lax.rem(kc, 2)

    def issue(slot_i, bb, kk):
        base = kk * KC
        for h in range(H):
            pltpu.make_async_copy(
                k_hbm.at[bb, pl.ds(base, KC), h, :],
                kbuf.at[slot_i, h], ksems.at[slot_i, h],
            ).start()
            pltpu.make_async_copy(
                v_hbm.at[bb, pl.ds(base, KC), h, :],
                vbuf.at[slot_i, h], vsems.at[slot_i, h],
            ).start()

    @pl.when((b == 0) & (kc == 0))
    def _():
        issue(0, 0, 0)

    nxt = b * NKC + kc + 1

    @pl.when(nxt < B * NKC)
    def _():
        issue(1 - slot, nxt // NKC, lax.rem(nxt, NKC))

    @pl.when(kc == 0)
    def _():
        a_acc[...] = jnp.zeros_like(a_acc)
        r_acc[...] = jnp.zeros_like(r_acc)

    for h in range(H):
        pltpu.make_async_copy(
            k_hbm.at[b, pl.ds(0, KC), h, :], kbuf.at[slot, h], ksems.at[slot, h]
        ).wait()
        pltpu.make_async_copy(
            v_hbm.at[b, pl.ds(0, KC), h, :], vbuf.at[slot, h], vsems.at[slot, h]
        ).wait()

        q = (q_ref[0, :, h, :] * SCALE).astype(jnp.bfloat16)
        k = kbuf[slot, h].astype(jnp.bfloat16)
        v = vbuf[slot, h].astype(jnp.bfloat16)

        s = lax.dot_general(
            q, k, (((1,), (1,)), ((), ())),
            preferred_element_type=jnp.float32,
        )
        p = jnp.exp(s)
        r_acc[h, :, :] += jnp.sum(p, axis=1, keepdims=True)
        a_acc[h, :, :] += lax.dot_general(
            p.astype(jnp.bfloat16), v, (((1,), (0,)), ((), ())),
            preferred_element_type=jnp.float32,
        )

    @pl.when(kc == NKC - 1)
    def _():
        a_ref[0, :, :, :] = a_acc[...]
        r_ref[0, :, :, :] = r_acc[...]


def _flash_partial(Q, K, V):
    return pl.pallas_call(
        _flash_partial_body,
        grid=(B, NKC),
        in_specs=[
            pl.BlockSpec((1, SQ, H, D), lambda b, k: (b, 0, 0, 0)),
            pl.BlockSpec(memory_space=pltpu.MemorySpace.HBM),
            pl.BlockSpec(memory_space=pltpu.MemorySpace.HBM),
        ],
        out_specs=[
            pl.BlockSpec((1, H, SQ, D), lambda b, k: (b, 0, 0, 0)),
            pl.BlockSpec((1, H, SQ, 1), lambda b, k: (b, 0, 0, 0)),
        ],
        out_shape=[
            jax.ShapeDtypeStruct((B, H, SQ, D), jnp.float32),
            jax.ShapeDtypeStruct((B, H, SQ, 1), jnp.float32),
        ],
        scratch_shapes=[
            pltpu.VMEM((2, H, KC, D), jnp.float32),
            pltpu.VMEM((2, H, KC, D), jnp.float32),
            pltpu.SemaphoreType.DMA((2, H)),
            pltpu.SemaphoreType.DMA((2, H)),
            pltpu.VMEM((H, SQ, D), jnp.float32),
            pltpu.VMEM((H, SQ, 1), jnp.float32),
        ],
        compiler_params=pltpu.CompilerParams(
            dimension_semantics=("arbitrary", "arbitrary"),
        ),
    )(Q, K, V)


def _allreduce_body(x_ref, out_ref, comm_ref, send_buf, send_sems, recv_sems):
    my = lax.axis_index("i")
    p1 = my ^ 1
    p2 = 3 - my

    barrier_sem = pltpu.get_barrier_semaphore()
    for nbr in (p1, p2):
        pl.semaphore_signal(
            barrier_sem, inc=1,
            device_id=(nbr,), device_id_type=pl.DeviceIdType.MESH,
        )
    pl.semaphore_wait(barrier_sem, 2)

    rdma1 = pltpu.make_async_remote_copy(
        src_ref=x_ref,
        dst_ref=comm_ref.at[0],
        send_sem=send_sems.at[0],
        recv_sem=recv_sems.at[0],
        device_id=(p1,),
        device_id_type=pl.DeviceIdType.MESH,
    )
    rdma1.start()
    rdma1.wait()

    part = x_ref[...].astype(jnp.float32) + comm_ref[0].astype(jnp.float32)
    send_buf[...] = part.astype(jnp.bfloat16)

    rdma2 = pltpu.make_async_remote_copy(
        src_ref=send_buf,
        dst_ref=comm_ref.at[1],
        send_sem=send_sems.at[1],
        recv_sem=recv_sems.at[1],
        device_id=(p2,),
        device_id_type=pl.DeviceIdType.MESH,
    )
    rdma2.start()
    rdma2.wait()

    out_ref[...] = part + comm_ref[1].astype(jnp.float32)


def _allreduce(packed):
    return pl.pallas_call(
        _allreduce_body,
        in_specs=[pl.BlockSpec(memory_space=pltpu.VMEM)],
        out_specs=pl.BlockSpec(memory_space=pltpu.VMEM),
        out_shape=jax.ShapeDtypeStruct((PACKED_ROWS, D), jnp.float32),
        scratch_shapes=[
            pltpu.VMEM((2, PACKED_ROWS, D), jnp.bfloat16),
            pltpu.VMEM((PACKED_ROWS, D), jnp.bfloat16),
            pltpu.SemaphoreType.DMA((2,)),
            pltpu.SemaphoreType.DMA((2,)),
        ],
        compiler_params=pltpu.CompilerParams(collective_id=0),
    )(packed)


def kernel(Q, K, V):
    A, r = _flash_partial(Q, K, V)
    packed = jnp.concatenate(
        [A.reshape(ROWS, D), r.reshape(ROWS // D, D)], axis=0
    ).astype(jnp.bfloat16)
    red = _allreduce(packed)
    o = red[:ROWS] / red[ROWS:].reshape(ROWS, 1)
    return o.reshape(B, H, SQ, D).transpose(0, 2, 1, 3)
